# Initial kernel scaffold; baseline (speedup 1.0000x reference)
import jax
import jax.numpy as jnp
from jax.experimental import pallas as pl


def kernel(x, edge_index, env_edge_attr, act_edge_attr, history, Wc, bc, W_root, W_msg, b_msg):
    raise NotImplementedError("write your pallas kernel here")


# speedup vs baseline: 2.9840x; 2.9840x over previous
import functools

import jax
import jax.numpy as jnp
from jax import lax
from jax.experimental import pallas as pl
from jax.experimental.pallas import tpu as pltpu
from jax.experimental.pallas import tpu_sc as plsc

NC = 2
NS = 16
NW = NC * NS
CW = 125


def _film_tc(x_ref, hist_ref, wc_ref, bc_ref, wmx_ref, wr_ref, h_ref, r_ref):
    d = x_ref.shape[1]
    cond = jnp.dot(hist_ref[...], wc_ref[...], preferred_element_type=jnp.float32)
    cond = cond + bc_ref[...]
    x0 = cond[:, :d] * x_ref[...] + cond[:, d:]
    h_ref[...] = jnp.dot(x0, wmx_ref[...], preferred_element_type=jnp.float32)
    r_ref[...] = jnp.dot(x0, wr_ref[...], preferred_element_type=jnp.float32)


def _mid_tc(r_ref, agg_ref, b_ref, wmx_ref, wr_ref, h_ref, rout_ref):
    n = r_ref.shape[0]
    xl = jnp.maximum(
        r_ref[...] + agg_ref[0, :n, :] + agg_ref[1, :n, :] + b_ref[...], 0.0)
    h_ref[...] = jnp.dot(xl, wmx_ref[...], preferred_element_type=jnp.float32)
    rout_ref[...] = jnp.dot(xl, wr_ref[...], preferred_element_type=jnp.float32)


def _final_tc(r_ref, agg_ref, b_ref, out_ref):
    n = r_ref.shape[0]
    out_ref[...] = r_ref[...] + agg_ref[0, :n, :] + agg_ref[1, :n, :] + b_ref[...]


def _eproj_tc(ea_ref, wme_ref, out_ref):
    out_ref[...] = jnp.dot(ea_ref[...], wme_ref[...], preferred_element_type=jnp.float32)


def _sc_edge_body(h_hbm, ep_hbm, src_hbm, dst_hbm, zeros_hbm, out_hbm,
                  sidx, didx, rows, ebuf, sem, agg):
    ch = src_hbm.shape[1]
    rows_per_tile = agg.shape[0] // NS
    c = lax.axis_index("c")
    s = lax.axis_index("s")
    wid = c * NS + s

    pltpu.sync_copy(zeros_hbm, agg.at[pl.ds(s * rows_per_tile, rows_per_tile)])
    plsc.subcore_barrier()

    def outer(jo, carry):
        pltpu.sync_copy(src_hbm.at[wid, pl.ds(jo * 8, 8)], sidx)
        pltpu.sync_copy(dst_hbm.at[wid, pl.ds(jo * 8, 8)], didx)

        def chunk(ji, carry2):
            j = jo * 8 + ji
            pltpu.sync_copy(ep_hbm.at[wid, j], ebuf)
            pltpu.async_copy(h_hbm.at[sidx.at[ji]], rows, sem).wait()

            def row(i, carry3):
                for k in range(8):
                    sl = pl.ds(k * 16, 16)
                    rows[i, sl] = jnp.maximum(rows[i, sl] + ebuf[i, sl], 0.0)
                return carry3

            lax.fori_loop(0, CW, row, 0)
            pltpu.sync_copy(rows, agg.at[didx.at[ji]], add=True)
            return carry2

        lax.fori_loop(0, 8, chunk, 0)
        return carry

    lax.fori_loop(0, ch // 8, outer, 0)
    plsc.subcore_barrier()

    pltpu.sync_copy(agg.at[pl.ds(s * rows_per_tile, rows_per_tile)],
                    out_hbm.at[c, pl.ds(s * rows_per_tile, rows_per_tile)])


def _make_sc_call(n_pad, d, ch):
    mesh = plsc.VectorSubcoreMesh(core_axis_name="c", subcore_axis_name="s",
                                  num_cores=NC, num_subcores=NS)
    return pl.kernel(
        _sc_edge_body,
        out_type=jax.ShapeDtypeStruct((NC, n_pad, d), jnp.float32),
        mesh=mesh,
        scratch_types=[
            pltpu.VMEM((8, CW), jnp.int32),
            pltpu.VMEM((8, CW), jnp.int32),
            pltpu.VMEM((CW, d), jnp.float32),
            pltpu.VMEM((CW, d), jnp.float32),
            pltpu.SemaphoreType.DMA,
            pltpu.VMEM_SHARED((n_pad, d), jnp.float32),
        ],
    )


def kernel(x, edge_index, env_edge_attr, act_edge_attr, history, Wc, bc,
           W_root, W_msg, b_msg):
    n, d = x.shape
    e = edge_index.shape[1]
    L = W_root.shape[0]
    de = env_edge_attr.shape[1]
    ch = e // (NW * CW)
    n_pad = ((n + NS * 8 - 1) // (NS * 8)) * NS * 8
    rows_per_tile = n_pad // NS

    src = edge_index[0].reshape(NW, ch, CW)
    dst = edge_index[1].reshape(NW, ch, CW)
    zeros = jnp.zeros((rows_per_tile, d), jnp.float32)
    bc2 = bc.reshape(1, 2 * d)

    full = lambda shape: pl.BlockSpec(shape, lambda: (0,) * len(shape))

    film = pl.pallas_call(
        _film_tc,
        out_shape=[jax.ShapeDtypeStruct((n, d), jnp.float32)] * 2,
        in_specs=[full((n, d)), full((n, d)), full((d, 2 * d)), full((1, 2 * d)),
                  full((d, d)), full((d, d))],
        out_specs=[full((n, d)), full((n, d))],
    )

    mid = pl.pallas_call(
        _mid_tc,
        out_shape=[jax.ShapeDtypeStruct((n, d), jnp.float32)] * 2,
        in_specs=[full((n, d)), full((NC, n_pad, d)), full((1, d)),
                  full((d, d)), full((d, d))],
        out_specs=[full((n, d)), full((n, d))],
    )

    final = pl.pallas_call(
        _final_tc,
        out_shape=jax.ShapeDtypeStruct((n, d), jnp.float32),
        in_specs=[full((n, d)), full((NC, n_pad, d)), full((1, d))],
        out_specs=full((n, d)),
    )

    eb = 8000
    eproj = pl.pallas_call(
        _eproj_tc,
        grid=(e // eb,),
        out_shape=jax.ShapeDtypeStruct((e, d), jnp.float32),
        in_specs=[pl.BlockSpec((eb, de), lambda i: (i, 0)),
                  pl.BlockSpec((de, d), lambda i: (0, 0))],
        out_specs=pl.BlockSpec((eb, d), lambda i: (i, 0)),
    )

    sc_call = _make_sc_call(n_pad, d, ch)

    edge_attrs = [env_edge_attr] + [act_edge_attr] * (L - 1)

    h, r = film(x, history, Wc, bc2, W_msg[0][:d], W_root[0])
    for l in range(L):
        ep = eproj(edge_attrs[l], W_msg[l][d:]).reshape(NW, ch, CW, d)
        aggp = sc_call(h, ep, src, dst, zeros)
        bl = b_msg[l].reshape(1, d)
        if l + 1 < L:
            h, r = mid(r, aggp, bl, W_msg[l + 1][:d], W_root[l + 1])
        else:
            out = final(r, aggp, bl)
    return out


# speedup vs baseline: 4.4818x; 1.5020x over previous
import functools

import jax
import jax.numpy as jnp
from jax import lax
from jax.experimental import pallas as pl
from jax.experimental.pallas import tpu as pltpu
from jax.experimental.pallas import tpu_sc as plsc

NC = 2
NS = 16
NW = NC * NS
CW = 80


def _film_tc(x_ref, hist_ref, wc_ref, bc_ref, wmx_ref, wr_ref, h_ref, r_ref):
    d = x_ref.shape[1]
    cond = jnp.dot(hist_ref[...], wc_ref[...], preferred_element_type=jnp.float32)
    cond = cond + bc_ref[...]
    x0 = cond[:, :d] * x_ref[...] + cond[:, d:]
    h_ref[...] = jnp.dot(x0, wmx_ref[...], preferred_element_type=jnp.float32)
    r_ref[...] = jnp.dot(x0, wr_ref[...], preferred_element_type=jnp.float32)


def _mid_tc(r_ref, agg_ref, b_ref, wmx_ref, wr_ref, h_ref, rout_ref):
    n = r_ref.shape[0]
    xl = jnp.maximum(
        r_ref[...] + agg_ref[0, :n, :] + agg_ref[1, :n, :] + b_ref[...], 0.0)
    h_ref[...] = jnp.dot(xl, wmx_ref[...], preferred_element_type=jnp.float32)
    rout_ref[...] = jnp.dot(xl, wr_ref[...], preferred_element_type=jnp.float32)


def _final_tc(r_ref, agg_ref, b_ref, out_ref):
    n = r_ref.shape[0]
    out_ref[...] = r_ref[...] + agg_ref[0, :n, :] + agg_ref[1, :n, :] + b_ref[...]


def _eproj_tc(ea_ref, wme_ref, out_ref):
    out_ref[...] = jnp.dot(ea_ref[...], wme_ref[...], preferred_element_type=jnp.float32)


def _sc_edge_body(h_hbm, ep_hbm, src_hbm, dst_hbm, zeros_hbm, out_hbm,
                  sidx, didx, rows0, rows1, ebuf0, ebuf1,
                  gsem0, gsem1, esem0, esem1, agg):
    nch = ep_hbm.shape[1] // CW
    nb_full = nch // 8
    tail = nch - nb_full * 8
    rows_per_tile = agg.shape[0] // NS
    c = lax.axis_index("c")
    s = lax.axis_index("s")
    wid = c * NS + s

    rows_ = (rows0, rows1)
    ebuf_ = (ebuf0, ebuf1)
    gsem_ = (gsem0, gsem1)
    esem_ = (esem0, esem1)

    pltpu.sync_copy(zeros_hbm, agg.at[pl.ds(s * rows_per_tile, rows_per_tile)])
    plsc.subcore_barrier()

    def run_block(j0, ji0, count):
        pend = {}

        def issue(j, ji, b):
            e = pltpu.async_copy(ep_hbm.at[wid, pl.ds(j * CW, CW)],
                                 ebuf_[b], esem_[b])
            g = pltpu.async_copy(h_hbm.at[sidx.at[ji]], rows_[b], gsem_[b])
            pend[b] = (g, e)

        issue(j0 + ji0, ji0, 0)
        for i in range(count):
            b = i % 2
            if i + 1 < count:
                issue(j0 + ji0 + i + 1, ji0 + i + 1, 1 - b)
            g, e = pend[b]
            g.wait()
            e.wait()

            def row(r, carry):
                for k in range(8):
                    sl = pl.ds(k * 16, 16)
                    rows_[b][r, sl] = jnp.maximum(
                        rows_[b][r, sl] + ebuf_[b][r, sl], 0.0)
                return carry

            lax.fori_loop(0, CW, row, 0)
            pltpu.sync_copy(rows_[b], agg.at[didx.at[ji0 + i]], add=True)

    def block(jo, carry):
        pltpu.sync_copy(src_hbm.at[wid, pl.ds(jo * 8, 8)], sidx)
        pltpu.sync_copy(dst_hbm.at[wid, pl.ds(jo * 8, 8)], didx)
        run_block(jo * 8, 0, 8)
        return carry

    lax.fori_loop(0, nb_full, block, 0)

    if tail:
        pltpu.sync_copy(src_hbm.at[wid, pl.ds(nb_full * 8, 8)], sidx)
        pltpu.sync_copy(dst_hbm.at[wid, pl.ds(nb_full * 8, 8)], didx)
        run_block(nb_full * 8, 0, tail)

    plsc.subcore_barrier()

    pltpu.sync_copy(agg.at[pl.ds(s * rows_per_tile, rows_per_tile)],
                    out_hbm.at[c, pl.ds(s * rows_per_tile, rows_per_tile)])


def _make_sc_call(n_pad, d, ch):
    mesh = plsc.VectorSubcoreMesh(core_axis_name="c", subcore_axis_name="s",
                                  num_cores=NC, num_subcores=NS)
    return pl.kernel(
        _sc_edge_body,
        out_type=jax.ShapeDtypeStruct((NC, n_pad, d), jnp.float32),
        mesh=mesh,
        scratch_types=[
            pltpu.VMEM((8, CW), jnp.int32),
            pltpu.VMEM((8, CW), jnp.int32),
            pltpu.VMEM((CW, d), jnp.float32),
            pltpu.VMEM((CW, d), jnp.float32),
            pltpu.VMEM((CW, d), jnp.float32),
            pltpu.VMEM((CW, d), jnp.float32),
            pltpu.SemaphoreType.DMA,
            pltpu.SemaphoreType.DMA,
            pltpu.SemaphoreType.DMA,
            pltpu.SemaphoreType.DMA,
            pltpu.VMEM_SHARED((n_pad, d), jnp.float32),
        ],
    )


def kernel(x, edge_index, env_edge_attr, act_edge_attr, history, Wc, bc,
           W_root, W_msg, b_msg):
    n, d = x.shape
    e = edge_index.shape[1]
    L = W_root.shape[0]
    de = env_edge_attr.shape[1]
    ch = e // (NW * CW)
    ch_pad = ((ch + 7) // 8) * 8
    n_pad = ((n + NS * 8 - 1) // (NS * 8)) * NS * 8
    rows_per_tile = n_pad // NS

    src = jnp.pad(edge_index[0].reshape(NW, ch, CW),
                  ((0, 0), (0, ch_pad - ch), (0, 0)))
    dst = jnp.pad(edge_index[1].reshape(NW, ch, CW),
                  ((0, 0), (0, ch_pad - ch), (0, 0)))
    zeros = jnp.zeros((rows_per_tile, d), jnp.float32)
    bc2 = bc.reshape(1, 2 * d)

    full = lambda shape: pl.BlockSpec(shape, lambda: (0,) * len(shape))

    film = pl.pallas_call(
        _film_tc,
        out_shape=[jax.ShapeDtypeStruct((n, d), jnp.float32)] * 2,
        in_specs=[full((n, d)), full((n, d)), full((d, 2 * d)), full((1, 2 * d)),
                  full((d, d)), full((d, d))],
        out_specs=[full((n, d)), full((n, d))],
    )

    mid = pl.pallas_call(
        _mid_tc,
        out_shape=[jax.ShapeDtypeStruct((n, d), jnp.float32)] * 2,
        in_specs=[full((n, d)), full((NC, n_pad, d)), full((1, d)),
                  full((d, d)), full((d, d))],
        out_specs=[full((n, d)), full((n, d))],
    )

    final = pl.pallas_call(
        _final_tc,
        out_shape=jax.ShapeDtypeStruct((n, d), jnp.float32),
        in_specs=[full((n, d)), full((NC, n_pad, d)), full((1, d))],
        out_specs=full((n, d)),
    )

    eb = 8000
    eproj = pl.pallas_call(
        _eproj_tc,
        grid=(e // eb,),
        out_shape=jax.ShapeDtypeStruct((e, d), jnp.float32),
        in_specs=[pl.BlockSpec((eb, de), lambda i: (i, 0)),
                  pl.BlockSpec((de, d), lambda i: (0, 0))],
        out_specs=pl.BlockSpec((eb, d), lambda i: (i, 0)),
    )

    sc_call = _make_sc_call(n_pad, d, ch)

    edge_attrs = [env_edge_attr] + [act_edge_attr] * (L - 1)

    h, r = film(x, history, Wc, bc2, W_msg[0][:d], W_root[0])
    for l in range(L):
        ep = eproj(edge_attrs[l], W_msg[l][d:]).reshape(NW, ch * CW, d)
        aggp = sc_call(h, ep, src, dst, zeros)
        bl = b_msg[l].reshape(1, d)
        if l + 1 < L:
            h, r = mid(r, aggp, bl, W_msg[l + 1][:d], W_root[l + 1])
        else:
            out = final(r, aggp, bl)
    return out


# speedup vs baseline: 4.5931x; 1.0248x over previous
import functools

import jax
import jax.numpy as jnp
from jax import lax
from jax.experimental import pallas as pl
from jax.experimental.pallas import tpu as pltpu
from jax.experimental.pallas import tpu_sc as plsc

NC = 2
NS = 16
NW = NC * NS
CW = 80


def _film_tc(x_ref, hist_ref, wc_ref, bc_ref, wmx_ref, wr_ref, h_ref, r_ref):
    d = x_ref.shape[1]
    cond = jnp.dot(hist_ref[...], wc_ref[...], preferred_element_type=jnp.float32)
    cond = cond + bc_ref[...]
    x0 = cond[:, :d] * x_ref[...] + cond[:, d:]
    h_ref[...] = jnp.dot(x0, wmx_ref[...], preferred_element_type=jnp.float32)
    r_ref[...] = jnp.dot(x0, wr_ref[...], preferred_element_type=jnp.float32)


def _mid_tc(r_ref, agg_ref, b_ref, wmx_ref, wr_ref, h_ref, rout_ref):
    n = r_ref.shape[0]
    xl = jnp.maximum(
        r_ref[...] + agg_ref[0, :n, :] + agg_ref[1, :n, :] + b_ref[...], 0.0)
    h_ref[...] = jnp.dot(xl, wmx_ref[...], preferred_element_type=jnp.float32)
    rout_ref[...] = jnp.dot(xl, wr_ref[...], preferred_element_type=jnp.float32)


def _final_tc(r_ref, agg_ref, b_ref, out_ref):
    n = r_ref.shape[0]
    out_ref[...] = r_ref[...] + agg_ref[0, :n, :] + agg_ref[1, :n, :] + b_ref[...]


def _eproj_tc(ea_ref, wme_ref, out_ref):
    d = wme_ref.shape[1]
    m = jnp.dot(ea_ref[...], wme_ref[...], preferred_element_type=jnp.float32)
    u = lax.bitcast_convert_type(m.astype(jnp.bfloat16), jnp.uint16)
    u = u.astype(jnp.uint32)
    packed = u[:, : d // 2] | (u[:, d // 2:] << 16)
    out_ref[...] = lax.bitcast_convert_type(packed, jnp.int32)


def _sc_edge_body(h_hbm, ep_hbm, src_hbm, dst_hbm, zeros_hbm, out_hbm,
                  sidx, didx, rows0, rows1, ebuf0, ebuf1,
                  gsem0, gsem1, esem0, esem1, agg):
    nch = ep_hbm.shape[1] // CW
    nb_full = nch // 8
    tail = nch - nb_full * 8
    rows_per_tile = agg.shape[0] // NS
    c = lax.axis_index("c")
    s = lax.axis_index("s")
    wid = c * NS + s

    rows_ = (rows0, rows1)
    ebuf_ = (ebuf0, ebuf1)
    gsem_ = (gsem0, gsem1)
    esem_ = (esem0, esem1)

    pltpu.sync_copy(zeros_hbm, agg.at[pl.ds(s * rows_per_tile, rows_per_tile)])
    plsc.subcore_barrier()

    def run_block(j0, ji0, count):
        pend = {}

        def issue(j, ji, b):
            e = pltpu.async_copy(ep_hbm.at[wid, pl.ds(j * CW, CW)],
                                 ebuf_[b], esem_[b])
            g = pltpu.async_copy(h_hbm.at[sidx.at[ji]], rows_[b], gsem_[b])
            pend[b] = (g, e)

        issue(j0 + ji0, ji0, 0)
        for i in range(count):
            b = i % 2
            if i + 1 < count:
                issue(j0 + ji0 + i + 1, ji0 + i + 1, 1 - b)
            g, e = pend[b]
            g.wait()
            e.wait()

            def row(r, carry):
                half = rows_[b].shape[1] // 2
                for m in range(half // 16):
                    v = ebuf_[b][r, pl.ds(m * 16, 16)]
                    lo = lax.bitcast_convert_type(v << 16, jnp.float32)
                    hi = lax.bitcast_convert_type((v >> 16) << 16, jnp.float32)
                    sl = pl.ds(m * 16, 16)
                    sh = pl.ds(half + m * 16, 16)
                    rows_[b][r, sl] = jnp.maximum(rows_[b][r, sl] + lo, 0.0)
                    rows_[b][r, sh] = jnp.maximum(rows_[b][r, sh] + hi, 0.0)
                return carry

            lax.fori_loop(0, CW, row, 0)
            pltpu.sync_copy(rows_[b], agg.at[didx.at[ji0 + i]], add=True)

    def block(jo, carry):
        pltpu.sync_copy(src_hbm.at[wid, pl.ds(jo * 8, 8)], sidx)
        pltpu.sync_copy(dst_hbm.at[wid, pl.ds(jo * 8, 8)], didx)
        run_block(jo * 8, 0, 8)
        return carry

    lax.fori_loop(0, nb_full, block, 0)

    if tail:
        pltpu.sync_copy(src_hbm.at[wid, pl.ds(nb_full * 8, 8)], sidx)
        pltpu.sync_copy(dst_hbm.at[wid, pl.ds(nb_full * 8, 8)], didx)
        run_block(nb_full * 8, 0, tail)

    plsc.subcore_barrier()

    pltpu.sync_copy(agg.at[pl.ds(s * rows_per_tile, rows_per_tile)],
                    out_hbm.at[c, pl.ds(s * rows_per_tile, rows_per_tile)])


def _make_sc_call(n_pad, d, ch):
    mesh = plsc.VectorSubcoreMesh(core_axis_name="c", subcore_axis_name="s",
                                  num_cores=NC, num_subcores=NS)
    return pl.kernel(
        _sc_edge_body,
        out_type=jax.ShapeDtypeStruct((NC, n_pad, d), jnp.float32),
        mesh=mesh,
        scratch_types=[
            pltpu.VMEM((8, CW), jnp.int32),
            pltpu.VMEM((8, CW), jnp.int32),
            pltpu.VMEM((CW, d), jnp.float32),
            pltpu.VMEM((CW, d), jnp.float32),
            pltpu.VMEM((CW, d // 2), jnp.int32),
            pltpu.VMEM((CW, d // 2), jnp.int32),
            pltpu.SemaphoreType.DMA,
            pltpu.SemaphoreType.DMA,
            pltpu.SemaphoreType.DMA,
            pltpu.SemaphoreType.DMA,
            pltpu.VMEM_SHARED((n_pad, d), jnp.float32),
        ],
    )


def kernel(x, edge_index, env_edge_attr, act_edge_attr, history, Wc, bc,
           W_root, W_msg, b_msg):
    n, d = x.shape
    e = edge_index.shape[1]
    L = W_root.shape[0]
    de = env_edge_attr.shape[1]
    ch = e // (NW * CW)
    ch_pad = ((ch + 7) // 8) * 8
    n_pad = ((n + NS * 8 - 1) // (NS * 8)) * NS * 8
    rows_per_tile = n_pad // NS

    src = jnp.pad(edge_index[0].reshape(NW, ch, CW),
                  ((0, 0), (0, ch_pad - ch), (0, 0)))
    dst = jnp.pad(edge_index[1].reshape(NW, ch, CW),
                  ((0, 0), (0, ch_pad - ch), (0, 0)))
    zeros = jnp.zeros((rows_per_tile, d), jnp.float32)
    bc2 = bc.reshape(1, 2 * d)

    full = lambda shape: pl.BlockSpec(shape, lambda: (0,) * len(shape))

    film = pl.pallas_call(
        _film_tc,
        out_shape=[jax.ShapeDtypeStruct((n, d), jnp.float32)] * 2,
        in_specs=[full((n, d)), full((n, d)), full((d, 2 * d)), full((1, 2 * d)),
                  full((d, d)), full((d, d))],
        out_specs=[full((n, d)), full((n, d))],
    )

    mid = pl.pallas_call(
        _mid_tc,
        out_shape=[jax.ShapeDtypeStruct((n, d), jnp.float32)] * 2,
        in_specs=[full((n, d)), full((NC, n_pad, d)), full((1, d)),
                  full((d, d)), full((d, d))],
        out_specs=[full((n, d)), full((n, d))],
    )

    final = pl.pallas_call(
        _final_tc,
        out_shape=jax.ShapeDtypeStruct((n, d), jnp.float32),
        in_specs=[full((n, d)), full((NC, n_pad, d)), full((1, d))],
        out_specs=full((n, d)),
    )

    eb = 8000
    eproj = pl.pallas_call(
        _eproj_tc,
        grid=(e // eb,),
        out_shape=jax.ShapeDtypeStruct((e, d // 2), jnp.int32),
        in_specs=[pl.BlockSpec((eb, de), lambda i: (i, 0)),
                  pl.BlockSpec((de, d), lambda i: (0, 0))],
        out_specs=pl.BlockSpec((eb, d // 2), lambda i: (i, 0)),
    )

    sc_call = _make_sc_call(n_pad, d, ch)

    edge_attrs = [env_edge_attr] + [act_edge_attr] * (L - 1)

    h, r = film(x, history, Wc, bc2, W_msg[0][:d], W_root[0])
    for l in range(L):
        ep = eproj(edge_attrs[l], W_msg[l][d:]).reshape(NW, ch * CW, d // 2)
        aggp = sc_call(h, ep, src, dst, zeros)
        bl = b_msg[l].reshape(1, d)
        if l + 1 < L:
            h, r = mid(r, aggp, bl, W_msg[l + 1][:d], W_root[l + 1])
        else:
            out = final(r, aggp, bl)
    return out
